# baseline (speedup 1.0000x reference)
import functools

import jax
import jax.numpy as jnp
from jax import lax
from jax.experimental import pallas as pl
from jax.experimental.pallas import tpu as pltpu
from jax.experimental.pallas import tpu_sc as plsc

NUM_CORES = 2
NUM_SUBCORES = 16
LANES = 16
NW = NUM_CORES * NUM_SUBCORES

BATCH = 16384
EMBED_DIM = 64
BW = BATCH // NW
CHUNK = 128
NCHUNK = BW // CHUNK


def _body(pos_hbm, neg_hbm, table_hbm, out_hbm,
          idx_pos_v, idx_neg_v, rows_pos_v, rows_neg_v, trans_v, out_v, sem):
    wid = lax.axis_index("s") * NUM_CORES + lax.axis_index("c")

    pltpu.sync_copy(pos_hbm.at[pl.ds(wid * NCHUNK, NCHUNK)], idx_pos_v)
    pltpu.sync_copy(neg_hbm.at[pl.ds(wid * NCHUNK, NCHUNK)], idx_neg_v)

    copies = []
    for j in range(NCHUNK):
        copies.append(pltpu.async_copy(
            table_hbm.at[idx_pos_v.at[j]],
            rows_pos_v.at[pl.ds(j * CHUNK, CHUNK)], sem))
        copies.append(pltpu.async_copy(
            table_hbm.at[idx_neg_v.at[j]],
            rows_neg_v.at[pl.ds(j * CHUNK, CHUNK)], sem))
    for c in copies:
        c.wait()

    lane = lax.iota(jnp.int32, LANES)
    nchunks_d = EMBED_DIM // LANES

    def group(g, carry):
        r = g * LANES
        for i in range(LANES):
            b = r + i
            p = jnp.zeros((LANES,), jnp.float32)
            for j in range(nchunks_d):
                a_v = rows_pos_v[b, pl.ds(j * LANES, LANES)]
                b_v = rows_neg_v[b, pl.ds(j * LANES, LANES)]
                p = p + a_v * b_v
            plsc.store_scatter(trans_v, [lane * LANES + i], p)
        acc = trans_v[pl.ds(0, LANES)]
        for l in range(1, LANES):
            acc = acc + trans_v[pl.ds(l * LANES, LANES)]
        out_v[pl.ds(r, LANES)] = acc
        return carry

    lax.fori_loop(0, BW // LANES, group, 0)

    pltpu.sync_copy(out_v, out_hbm.at[pl.ds(wid * BW, BW)])


_sc_call = functools.partial(
    pl.kernel,
    mesh=plsc.VectorSubcoreMesh(core_axis_name="c", subcore_axis_name="s"),
    out_type=jax.ShapeDtypeStruct((BATCH,), jnp.float32),
    compiler_params=pltpu.CompilerParams(
        use_tc_tiling_on_sc=False, needs_layout_passes=False),
    scratch_types=[
        pltpu.VMEM((NCHUNK, CHUNK), jnp.int32),
        pltpu.VMEM((NCHUNK, CHUNK), jnp.int32),
        pltpu.VMEM((BW, EMBED_DIM), jnp.float32),
        pltpu.VMEM((BW, EMBED_DIM), jnp.float32),
        pltpu.VMEM((LANES * LANES,), jnp.float32),
        pltpu.VMEM((BW,), jnp.float32),
        pltpu.SemaphoreType.DMA,
    ],
)(_body)


@jax.jit
def kernel(stock_pos, stock_neg, embeddings):
    pos = stock_pos.astype(jnp.int32).reshape(NW * NCHUNK, CHUNK)
    neg = stock_neg.astype(jnp.int32).reshape(NW * NCHUNK, CHUNK)
    return _sc_call(pos, neg, embeddings)
